# Initial kernel scaffold; baseline (speedup 1.0000x reference)
import jax
import jax.numpy as jnp
from jax.experimental import pallas as pl


def kernel(x, enc_w1, enc_b1, enc_w2, enc_b2, enc_w3, enc_b3, codebook, dec_w1, dec_b1, dec_w2, dec_b2, dec_w3, dec_b3):
    raise NotImplementedError("write your pallas kernel here")


# speedup vs baseline: 1.0499x; 1.0499x over previous
import functools

import jax
import jax.numpy as jnp
from jax import lax
from jax.experimental import pallas as pl

K = 256
D = 256
N_FLAT = 25088
BLOCK_R = 3584
GRID = N_FLAT // BLOCK_R


def _vq_body(x_ref, cb_ref, q_ref, loss_ref, cnt_ref):
    i = pl.program_id(0)
    xb = x_ref[:, :]
    cb = cb_ref[:, :]
    dot = lax.dot_general(xb, cb, (((1,), (1,)), ((), ())),
                          preferred_element_type=jnp.float32)
    rowsq = jnp.sum(xb * xb, axis=1, keepdims=True)
    csq = jnp.sum(cb * cb, axis=1)
    dist = (rowsq + csq[None, :]) - 2.0 * dot
    dmin = jnp.min(dist, axis=1, keepdims=True)
    col = lax.broadcasted_iota(jnp.int32, dist.shape, 1)
    idx = jnp.min(jnp.where(dist == dmin, col, jnp.int32(K)), axis=1,
                  keepdims=True)
    onehot = (col == idx).astype(jnp.float32)
    q = lax.dot_general(onehot, cb, (((1,), (0,)), ((), ())),
                        preferred_element_type=jnp.float32)
    q_ref[:, :] = q
    diff = q - xb

    @pl.when(i == 0)
    def _init():
        loss_ref[:, :] = jnp.zeros((1, 1), jnp.float32)
        cnt_ref[:, :] = jnp.zeros((1, K), jnp.float32)

    loss_ref[:, :] += jnp.sum(diff * diff).reshape(1, 1)
    cnt_ref[:, :] += jnp.sum(onehot, axis=0).reshape(1, K)


@functools.partial(jax.jit, static_argnames=())
def _run_vq(flat, codebook):
    return pl.pallas_call(
        _vq_body,
        grid=(GRID,),
        in_specs=[
            pl.BlockSpec((BLOCK_R, D), lambda i: (i, 0)),
            pl.BlockSpec((K, D), lambda i: (0, 0)),
        ],
        out_specs=[
            pl.BlockSpec((BLOCK_R, D), lambda i: (i, 0)),
            pl.BlockSpec((1, 1), lambda i: (0, 0)),
            pl.BlockSpec((1, K), lambda i: (0, 0)),
        ],
        out_shape=[
            jax.ShapeDtypeStruct((N_FLAT, D), jnp.float32),
            jax.ShapeDtypeStruct((1, 1), jnp.float32),
            jax.ShapeDtypeStruct((1, K), jnp.float32),
        ],
    )(flat, codebook)


def _conv(x, w, b, stride, padding):
    y = lax.conv_general_dilated(x, w, (stride, stride),
                                 ((padding, padding), (padding, padding)),
                                 dimension_numbers=('NCHW', 'OIHW', 'NCHW'))
    return y + b[None, :, None, None]


def _conv_t(x, w, b, stride, padding):
    k = w.shape[2]
    w_t = jnp.transpose(jnp.flip(w, (2, 3)), (1, 0, 2, 3))
    pad = k - 1 - padding
    y = lax.conv_general_dilated(x, w_t, (1, 1),
                                 ((pad, pad), (pad, pad)),
                                 lhs_dilation=(stride, stride),
                                 dimension_numbers=('NCHW', 'OIHW', 'NCHW'))
    return y + b[None, :, None, None]


def kernel(x, enc_w1, enc_b1, enc_w2, enc_b2, enc_w3, enc_b3, codebook,
           dec_w1, dec_b1, dec_w2, dec_b2, dec_w3, dec_b3):
    h = jax.nn.relu(_conv(x, enc_w1, enc_b1, 2, 1))
    h = jax.nn.relu(_conv(h, enc_w2, enc_b2, 2, 1))
    encoded = _conv(h, enc_w3, enc_b3, 1, 0)

    flat = encoded.reshape(-1, D)
    q_flat, loss_sum, counts = _run_vq(flat, codebook)
    quantized = q_flat.reshape(encoded.shape)
    m = loss_sum[0, 0] / flat.size
    vq_loss = m + 0.25 * m
    avg_probs = counts[0] / flat.shape[0]
    perplexity = jnp.exp(-jnp.sum(avg_probs * jnp.log(avg_probs + 1e-10)))

    d = jax.nn.relu(_conv_t(quantized, dec_w1, dec_b1, 2, 1))
    d = jax.nn.relu(_conv_t(d, dec_w2, dec_b2, 2, 1))
    decoded = jax.nn.sigmoid(_conv_t(d, dec_w3, dec_b3, 1, 0))
    return (decoded, encoded, quantized, vq_loss, perplexity)
